# baseline (speedup 1.0000x reference)
import functools

import jax
import jax.numpy as jnp
import numpy as np
from jax import lax
from jax.experimental import pallas as pl
from jax.experimental.pallas import tpu as pltpu
from jax.experimental.pallas import tpu_sc as plsc

BATCH = 16384
EMBED_DIM = 64
NUM_CORES = 2
NUM_SUBCORES = 16
NUM_WORKERS = NUM_CORES * NUM_SUBCORES
ROWS_PER_WORKER = BATCH // NUM_WORKERS
IDX_CHUNK = 128
NUM_CHUNKS = ROWS_PER_WORKER // IDX_CHUNK
LANES = 16
DCHUNKS = EMBED_DIM // LANES
PARTIAL_WORDS = ROWS_PER_WORKER * LANES


def _sc_partials_body(head_hbm, rel_hbm, tail_hbm, ent_hbm, relemb_hbm,
                      out_hbm, hidx, ridx, tidx, hrows, rrows, trows,
                      part_v, sem):
    wid = lax.axis_index("s") * NUM_CORES + lax.axis_index("c")

    pltpu.sync_copy(head_hbm.at[wid], hidx)
    pltpu.sync_copy(rel_hbm.at[wid], ridx)
    pltpu.sync_copy(tail_hbm.at[wid], tidx)

    copies = []
    for j in range(NUM_CHUNKS):
        rows = pl.ds(j * IDX_CHUNK, IDX_CHUNK)
        copies.append(pltpu.async_copy(ent_hbm.at[hidx.at[j]], hrows.at[rows], sem))
        copies.append(pltpu.async_copy(relemb_hbm.at[ridx.at[j]], rrows.at[rows], sem))
        copies.append(pltpu.async_copy(ent_hbm.at[tidx.at[j]], trows.at[rows], sem))
    for c in copies:
        c.wait()

    def body(i, carry):
        acc = (hrows[i, pl.ds(0, LANES)] * rrows[i, pl.ds(0, LANES)]
               * trows[i, pl.ds(0, LANES)])
        for c in range(1, DCHUNKS):
            sl = pl.ds(c * LANES, LANES)
            acc = acc + hrows[i, sl] * rrows[i, sl] * trows[i, sl]
        part_v[pl.ds(i * LANES, LANES)] = acc
        return carry

    lax.fori_loop(0, ROWS_PER_WORKER, body, 0)

    pltpu.sync_copy(part_v, out_hbm.at[pl.ds(wid * PARTIAL_WORDS, PARTIAL_WORDS)])


_sc_partials = functools.partial(
    pl.kernel,
    mesh=plsc.VectorSubcoreMesh(core_axis_name="c", subcore_axis_name="s"),
    out_type=jax.ShapeDtypeStruct((BATCH * LANES,), jnp.float32),
    scratch_types=[
        pltpu.VMEM((NUM_CHUNKS, IDX_CHUNK), jnp.int32),
        pltpu.VMEM((NUM_CHUNKS, IDX_CHUNK), jnp.int32),
        pltpu.VMEM((NUM_CHUNKS, IDX_CHUNK), jnp.int32),
        pltpu.VMEM((ROWS_PER_WORKER, EMBED_DIM), jnp.float32),
        pltpu.VMEM((ROWS_PER_WORKER, EMBED_DIM), jnp.float32),
        pltpu.VMEM((ROWS_PER_WORKER, EMBED_DIM), jnp.float32),
        pltpu.VMEM((PARTIAL_WORDS,), jnp.float32),
        pltpu.SemaphoreType.DMA,
    ],
    compiler_params=pltpu.CompilerParams(use_tc_tiling_on_sc=False),
)(_sc_partials_body)

_FOLD = np.zeros((128, 8), np.float32)
for _l in range(128):
    _FOLD[_l, _l // 16] = 1.0


def _softmax_body(x_ref, m_ref, o_ref):
    scores = jnp.dot(x_ref[...], m_ref[...],
                     preferred_element_type=jnp.float32,
                     precision=jax.lax.Precision.HIGHEST)
    m = jnp.max(scores)
    e = jnp.exp(scores - m)
    o_ref[...] = e * (1.0 / jnp.sum(e))


_softmax = pl.pallas_call(
    _softmax_body,
    out_shape=jax.ShapeDtypeStruct((BATCH // 8, 8), jnp.float32),
)


def kernel(head_ids, rel_ids, tail_ids, entity_embeddings, relation_embeddings):
    h = head_ids.astype(jnp.int32).reshape(NUM_WORKERS, NUM_CHUNKS, IDX_CHUNK)
    r = rel_ids.astype(jnp.int32).reshape(NUM_WORKERS, NUM_CHUNKS, IDX_CHUNK)
    t = tail_ids.astype(jnp.int32).reshape(NUM_WORKERS, NUM_CHUNKS, IDX_CHUNK)
    partials = _sc_partials(h, r, t, entity_embeddings, relation_embeddings)
    out = _softmax(partials.reshape(BATCH // 8, 128), jnp.asarray(_FOLD))
    return out.reshape(BATCH)


# speedup vs baseline: 2.4451x; 2.4451x over previous
import functools

import jax
import jax.numpy as jnp
from jax import lax
from jax.experimental import pallas as pl
from jax.experimental.pallas import tpu as pltpu
from jax.experimental.pallas import tpu_sc as plsc

BATCH = 16384
EMBED_DIM = 64
NUM_ENT = 1000000
NUM_REL = 1000
NUM_CORES = 2
NUM_SUBCORES = 16
ROWS_PER_TILE = BATCH // NUM_SUBCORES
DIMS_PER_CORE = EMBED_DIM // NUM_CORES
LANES = 16
IDX_CHUNK = 128
NUM_IDX_CHUNKS = ROWS_PER_TILE // IDX_CHUNK
GROUPS_PER_CHUNK = IDX_CHUNK // LANES


def _sc_partials_body(hid_hbm, rid_hbm, tid_hbm, entT_hbm, relT_hbm,
                      out_hbm, hid_v, rid_v, tid_v, relcol,
                      hbuf, tbuf, scores_v, spcol, sem_s, sem_g):
    cid = lax.axis_index("c")
    sid = lax.axis_index("s")

    pltpu.sync_copy(hid_hbm.at[sid], hid_v)
    pltpu.sync_copy(rid_hbm.at[sid], rid_v)
    pltpu.sync_copy(tid_hbm.at[sid], tid_v)

    def zinit(g, carry):
        scores_v[pl.ds(g * LANES, LANES)] = jnp.zeros((LANES,), jnp.float32)
        return carry
    lax.fori_loop(0, ROWS_PER_TILE // LANES, zinit, 0)

    def dim(d, carry):
        j = cid * DIMS_PER_CORE + d

        @pl.when(sid == 0)
        def _stage():
            pltpu.async_copy(entT_hbm.at[j], spcol, sem_s).wait()

        pltpu.sync_copy(relT_hbm.at[j], relcol)

        plsc.subcore_barrier()

        copies = []
        for k in range(NUM_IDX_CHUNKS):
            copies.append(pltpu.async_copy(spcol.at[hid_v.at[k]],
                                           hbuf.at[k], sem_g))
            copies.append(pltpu.async_copy(spcol.at[tid_v.at[k]],
                                           tbuf.at[k], sem_g))
        for c in copies:
            c.wait()

        for k in range(NUM_IDX_CHUNKS):
            for g in range(GROUPS_PER_CHUNK):
                sl = pl.ds(g * LANES, LANES)
                hv = hbuf[k, sl]
                tv = tbuf[k, sl]
                rv = plsc.load_gather(relcol, [rid_v[k, sl]])
                row0 = k * IDX_CHUNK + g * LANES
                scores_v[pl.ds(row0, LANES)] = (
                    scores_v[pl.ds(row0, LANES)] + hv * rv * tv)

        plsc.subcore_barrier()
        return carry

    lax.fori_loop(0, DIMS_PER_CORE, dim, 0)

    pltpu.sync_copy(scores_v, out_hbm.at[cid, sid])


_sc_partials = functools.partial(
    pl.kernel,
    mesh=plsc.VectorSubcoreMesh(core_axis_name="c", subcore_axis_name="s"),
    out_type=jax.ShapeDtypeStruct((NUM_CORES, NUM_SUBCORES, ROWS_PER_TILE),
                                  jnp.float32),
    scratch_types=[
        pltpu.VMEM((NUM_IDX_CHUNKS, IDX_CHUNK), jnp.int32),
        pltpu.VMEM((NUM_IDX_CHUNKS, IDX_CHUNK), jnp.int32),
        pltpu.VMEM((NUM_IDX_CHUNKS, IDX_CHUNK), jnp.int32),
        pltpu.VMEM((NUM_REL,), jnp.float32),
        pltpu.VMEM((NUM_IDX_CHUNKS, IDX_CHUNK), jnp.float32),
        pltpu.VMEM((NUM_IDX_CHUNKS, IDX_CHUNK), jnp.float32),
        pltpu.VMEM((ROWS_PER_TILE,), jnp.float32),
        pltpu.VMEM_SHARED((NUM_ENT,), jnp.float32),
        pltpu.SemaphoreType.DMA,
        pltpu.SemaphoreType.DMA,
    ],
    compiler_params=pltpu.CompilerParams(needs_layout_passes=False),
)(_sc_partials_body)


def _softmax_body(x_ref, o_ref):
    scores = x_ref[0] + x_ref[1]
    m = jnp.max(scores)
    e = jnp.exp(scores - m)
    o_ref[...] = e * (1.0 / jnp.sum(e))


_softmax = pl.pallas_call(
    _softmax_body,
    out_shape=jax.ShapeDtypeStruct((128, 128), jnp.float32),
)


def kernel(head_ids, rel_ids, tail_ids, entity_embeddings, relation_embeddings):
    hid = head_ids.astype(jnp.int32).reshape(NUM_SUBCORES, NUM_IDX_CHUNKS,
                                             IDX_CHUNK)
    rid = rel_ids.astype(jnp.int32).reshape(NUM_SUBCORES, NUM_IDX_CHUNKS,
                                            IDX_CHUNK)
    tid = tail_ids.astype(jnp.int32).reshape(NUM_SUBCORES, NUM_IDX_CHUNKS,
                                             IDX_CHUNK)
    entT = entity_embeddings.T
    relT = relation_embeddings.T
    partials = _sc_partials(hid, rid, tid, entT, relT)
    return _softmax(partials.reshape(2, 128, 128)).reshape(BATCH)
